# baseline (speedup 1.0000x reference)
import functools

import jax
import jax.numpy as jnp
from jax import lax
from jax.experimental import pallas as pl
from jax.experimental.pallas import tpu as pltpu
from jax.experimental.pallas import tpu_sc as plsc

N = 10000
E = 320000
D = 128

NC = 2
NS = 16
NW = NC * NS
NP = 10240
C = 32
SB = 8
TOTB = 320
NCHK = TOTB // SB
EB = NW * TOTB
L = 16
STRIPE = NP // NS


def _tc_proj_kernel(x_ref, wt_ref, b_ref, k_ref, q_ref, v_ref, s_ref):
  x = x_ref[...]
  outs = (k_ref, q_ref, v_ref, s_ref)
  for i, o_ref in enumerate(outs):
    y = jnp.dot(x, wt_ref[i], preferred_element_type=jnp.float32)
    y = y + b_ref[i][None, :]
    if i < 2:
      y = jnp.exp(-jnp.clip(y, -70.0, 70.0))
    o_ref[...] = y


def _tc_proj(xp, wt, b):
  br = 1024
  grid = (NP // br,)
  out = jax.ShapeDtypeStruct((NP, D), jnp.float32)
  return pl.pallas_call(
      _tc_proj_kernel,
      grid=grid,
      in_specs=[
          pl.BlockSpec((br, D), lambda i: (i, 0)),
          pl.BlockSpec((4, D, D), lambda i: (0, 0, 0)),
          pl.BlockSpec((4, D), lambda i: (0, 0)),
      ],
      out_specs=[pl.BlockSpec((br, D), lambda i: (i, 0))] * 4,
      out_shape=[out] * 4,
  )(xp, wt, b)


def _tc_combine_kernel(p_ref, s_ref, o_ref):
  o_ref[...] = p_ref[0] + p_ref[1] + s_ref[...]


def _tc_combine(part, skip):
  br = 1024
  return pl.pallas_call(
      _tc_combine_kernel,
      grid=(NP // br,),
      in_specs=[
          pl.BlockSpec((2, br, D), lambda i: (0, i, 0)),
          pl.BlockSpec((br, D), lambda i: (i, 0)),
      ],
      out_specs=pl.BlockSpec((br, D), lambda i: (i, 0)),
      out_shape=jax.ShapeDtypeStruct((NP, D), jnp.float32),
  )(part, skip)


def _sc_edge_kernel(ek_hbm, eq_hbm, v_hbm, src2, dst2,
                    part_hbm, sbuf, dbuf, scidx, ekb, eqb, vb, msg,
                    agg_sh, ssem, gsem, csem):
  sid = lax.axis_index("s")
  cid = lax.axis_index("c")
  wid = sid * NC + cid
  rbase = wid * TOTB

  zf = jnp.zeros((L,), jnp.float32)

  def zrow(r, _):
    for j in range(D // L):
      msg[0, r, pl.ds(j * L, L)] = zf
    return 0

  lax.fori_loop(0, C, zrow, 0)
  for s in range(STRIPE // C):
    pltpu.sync_copy(msg.at[0], agg_sh.at[pl.ds(sid * STRIPE + s * C, C)])
  plsc.subcore_barrier()

  pltpu.async_copy(src2.at[pl.ds(rbase, SB)], sbuf.at[0], ssem.at[0])
  pltpu.async_copy(dst2.at[pl.ds(rbase, SB)], dbuf.at[0], ssem.at[0])
  pltpu.async_copy(src2.at[pl.ds(rbase + SB, SB)], sbuf.at[1], ssem.at[1])
  pltpu.async_copy(dst2.at[pl.ds(rbase + SB, SB)], dbuf.at[1], ssem.at[1])

  magic = jnp.full((L,), 0x7EF127EA, jnp.int32)

  def gblk(g, _):
    p = lax.rem(g, 3)
    ck = g // SB
    pc = lax.rem(ck, 2)
    r = lax.rem(g, SB)

    @pl.when((r == 0) & (g < TOTB))
    def _():
      pltpu.make_async_copy(src2.at[pl.ds(0, SB)], sbuf.at[pc],
                            ssem.at[pc]).wait()
      pltpu.make_async_copy(dst2.at[pl.ds(0, SB)], dbuf.at[pc],
                            ssem.at[pc]).wait()

    @pl.when(g < TOTB)
    def _():
      pltpu.async_copy(ek_hbm.at[dbuf.at[pc, r]], ekb.at[p], gsem.at[p])
      pltpu.async_copy(eq_hbm.at[sbuf.at[pc, r]], eqb.at[p], gsem.at[p])
      pltpu.async_copy(v_hbm.at[sbuf.at[pc, r]], vb.at[p], gsem.at[p])

    @pl.when(g >= 2)
    def _():
      gp = g - 2
      pp = lax.rem(gp, 3)
      ckp = lax.rem(gp // SB, 2)
      rp = lax.rem(gp, SB)
      pltpu.make_async_copy(ek_hbm.at[dbuf.at[0, 0]], ekb.at[pp],
                            gsem.at[pp]).wait()
      pltpu.make_async_copy(eq_hbm.at[sbuf.at[0, 0]], eqb.at[pp],
                            gsem.at[pp]).wait()
      pltpu.make_async_copy(v_hbm.at[sbuf.at[0, 0]], vb.at[pp],
                            gsem.at[pp]).wait()

      @pl.when((rp == SB - 1) & (gp // SB + 2 < NCHK))
      def _():
        nck = gp // SB + 2
        npc = lax.rem(nck, 2)
        pltpu.async_copy(src2.at[pl.ds(rbase + nck * SB, SB)],
                         sbuf.at[npc], ssem.at[npc])
        pltpu.async_copy(dst2.at[pl.ds(rbase + nck * SB, SB)],
                         dbuf.at[npc], ssem.at[npc])

      @pl.when(gp >= 1)
      def _():
        pltpu.make_async_copy(msg.at[0], agg_sh.at[scidx.at[0]],
                              csem.at[0]).wait()

      for i in range(C // L):
        scidx[0, pl.ds(i * L, L)] = dbuf[ckp, rp, pl.ds(i * L, L)]

      JV = D // L

      def edge(e, _):
        eks = [ekb[pp, e, pl.ds(j * L, L)] for j in range(JV)]
        eqs = [eqb[pp, e, pl.ds(j * L, L)] for j in range(JV)]
        xs = [1.0 + jnp.minimum(a * q, 1e30) for a, q in zip(eks, eqs)]
        ys = [plsc.bitcast(magic - plsc.bitcast(x, jnp.int32), jnp.float32)
              for x in xs]
        for _ in range(2):
          ts = [x * y for x, y in zip(xs, ys)]
          us = [2.0 - t for t in ts]
          ys = [y * u for y, u in zip(ys, us)]
        vs = [vb[pp, e, pl.ds(j * L, L)] for j in range(JV)]
        ms = [y * v for y, v in zip(ys, vs)]
        for j in range(JV):
          msg[0, e, pl.ds(j * L, L)] = ms[j]
        return 0

      lax.fori_loop(0, C, edge, 0)

      pltpu.async_copy(msg.at[0], agg_sh.at[scidx.at[0]], csem.at[0],
                       add=True)

    return 0

  lax.fori_loop(0, TOTB + 2, gblk, 0)

  pltpu.make_async_copy(msg.at[0], agg_sh.at[scidx.at[0]], csem.at[0]).wait()
  plsc.subcore_barrier()

  pltpu.sync_copy(agg_sh.at[pl.ds(sid * STRIPE, STRIPE)],
                  part_hbm.at[cid, pl.ds(sid * STRIPE, STRIPE)])


def _sc_edge(ek, eq, v, src2, dst2):
  mesh = plsc.VectorSubcoreMesh(
      core_axis_name="c", subcore_axis_name="s",
      num_cores=NC, num_subcores=NS)
  f = functools.partial(
      pl.kernel,
      out_type=jax.ShapeDtypeStruct((NC, NP, D), jnp.float32),
      mesh=mesh,
      compiler_params=pltpu.CompilerParams(needs_layout_passes=False),
      scratch_types=[
          pltpu.VMEM((2, SB, C), jnp.int32),
          pltpu.VMEM((2, SB, C), jnp.int32),
          pltpu.VMEM((1, C), jnp.int32),
          pltpu.VMEM((3, C, D), jnp.float32),
          pltpu.VMEM((3, C, D), jnp.float32),
          pltpu.VMEM((3, C, D), jnp.float32),
          pltpu.VMEM((1, C, D), jnp.float32),
          pltpu.VMEM_SHARED((NP, D), jnp.float32),
          pltpu.SemaphoreType.DMA((2,)),
          pltpu.SemaphoreType.DMA((3,)),
          pltpu.SemaphoreType.DMA((1,)),
      ],
  )(_sc_edge_kernel)
  return f(ek, eq, v, src2, dst2)


@jax.jit
def kernel(x, edge_index, edge_attr, Wk, bk, Wq, bq, Wv, bv, Wskip, bias):
  del edge_attr
  xp = jnp.pad(x, ((0, NP - N), (0, 0)))
  wt = jnp.stack([Wk.T, Wq.T, Wv.T, Wskip.T])
  b = jnp.stack([bk, bq, bv, bias])
  ek, eq, v, skip = _tc_proj(xp, wt, b)
  src = edge_index[0].astype(jnp.int32)
  dst = edge_index[1].astype(jnp.int32)
  pad = EB * C - E
  src2 = jnp.pad(src, (0, pad)).reshape(EB, C)
  dst2 = jnp.pad(dst, (0, pad), constant_values=N).reshape(EB, C)
  part = _sc_edge(ek, eq, v, src2, dst2)
  out = _tc_combine(part, skip)
  return out[:N]


# speedup vs baseline: 1.0562x; 1.0562x over previous
import functools

import jax
import jax.numpy as jnp
from jax import lax
from jax.experimental import pallas as pl
from jax.experimental.pallas import tpu as pltpu
from jax.experimental.pallas import tpu_sc as plsc

N = 10000
E = 320000
D = 128

NC = 2
NS = 16
NW = NC * NS
NP = 10240
C = 32
SB = 16
TOTB = 320
NCHK = TOTB // SB
EB = NW * TOTB
L = 16
STRIPE = NP // NS


def _tc_proj_kernel(x_ref, wt_ref, b_ref, k_ref, q_ref, v_ref, s_ref):
  x = x_ref[...]
  outs = (k_ref, q_ref, v_ref, s_ref)
  for i, o_ref in enumerate(outs):
    y = jnp.dot(x, wt_ref[i], preferred_element_type=jnp.float32)
    y = y + b_ref[i][None, :]
    if i < 2:
      y = jnp.exp(-jnp.clip(y, -70.0, 70.0))
    o_ref[...] = y


def _tc_proj(xp, wt, b):
  br = 1024
  grid = (NP // br,)
  out = jax.ShapeDtypeStruct((NP, D), jnp.float32)
  return pl.pallas_call(
      _tc_proj_kernel,
      grid=grid,
      in_specs=[
          pl.BlockSpec((br, D), lambda i: (i, 0)),
          pl.BlockSpec((4, D, D), lambda i: (0, 0, 0)),
          pl.BlockSpec((4, D), lambda i: (0, 0)),
      ],
      out_specs=[pl.BlockSpec((br, D), lambda i: (i, 0))] * 4,
      out_shape=[out] * 4,
  )(xp, wt, b)


def _tc_combine_kernel(p_ref, s_ref, o_ref):
  o_ref[...] = p_ref[0] + p_ref[1] + s_ref[...]


def _tc_combine(part, skip):
  br = 1024
  return pl.pallas_call(
      _tc_combine_kernel,
      grid=(NP // br,),
      in_specs=[
          pl.BlockSpec((2, br, D), lambda i: (0, i, 0)),
          pl.BlockSpec((br, D), lambda i: (i, 0)),
      ],
      out_specs=pl.BlockSpec((br, D), lambda i: (i, 0)),
      out_shape=jax.ShapeDtypeStruct((NP, D), jnp.float32),
  )(part, skip)


def _sc_edge_kernel(ek_hbm, eq_hbm, v_hbm, src2, dst2,
                    part_hbm, sbuf, dbuf, scidx, ekb, eqb, vb, msg,
                    agg_sh, ssem, gsem, csem):
  sid = lax.axis_index("s")
  cid = lax.axis_index("c")
  wid = sid * NC + cid
  rbase = wid * TOTB

  zf = jnp.zeros((L,), jnp.float32)

  def zrow(r, _):
    for j in range(D // L):
      msg[0, r, pl.ds(j * L, L)] = zf
    return 0

  lax.fori_loop(0, C, zrow, 0)
  for s in range(STRIPE // C):
    pltpu.sync_copy(msg.at[0], agg_sh.at[pl.ds(sid * STRIPE + s * C, C)])
  plsc.subcore_barrier()

  pltpu.async_copy(src2.at[pl.ds(rbase, SB)], sbuf.at[0], ssem.at[0])
  pltpu.async_copy(dst2.at[pl.ds(rbase, SB)], dbuf.at[0], ssem.at[0])

  magic = jnp.full((L,), 0x7EF127EA, jnp.int32)

  def gblk(g, _):
    p = lax.rem(g, 2)
    ck = g // SB
    pc = lax.rem(ck, 2)
    r = lax.rem(g, SB)

    @pl.when((r == 0) & (g < TOTB))
    def _():
      pltpu.make_async_copy(src2.at[pl.ds(0, SB)], sbuf.at[pc],
                            ssem.at[pc]).wait()
      pltpu.make_async_copy(dst2.at[pl.ds(0, SB)], dbuf.at[pc],
                            ssem.at[pc]).wait()

    @pl.when(g < TOTB)
    def _():
      pltpu.async_copy(ek_hbm.at[dbuf.at[pc, r]], ekb.at[p], gsem.at[p])
      pltpu.async_copy(eq_hbm.at[sbuf.at[pc, r]], eqb.at[p], gsem.at[p])
      pltpu.async_copy(v_hbm.at[sbuf.at[pc, r]], vb.at[p], gsem.at[p])

    @pl.when(g > 0)
    def _():
      gp = g - 1
      pp = 1 - p
      ckp = lax.rem(gp // SB, 2)
      rp = lax.rem(gp, SB)
      pltpu.make_async_copy(ek_hbm.at[dbuf.at[0, 0]], ekb.at[pp],
                            gsem.at[pp]).wait()
      pltpu.make_async_copy(eq_hbm.at[sbuf.at[0, 0]], eqb.at[pp],
                            gsem.at[pp]).wait()
      pltpu.make_async_copy(v_hbm.at[sbuf.at[0, 0]], vb.at[pp],
                            gsem.at[pp]).wait()

      @pl.when((r == 1) & (ck + 1 < NCHK))
      def _():
        pltpu.async_copy(src2.at[pl.ds(rbase + (ck + 1) * SB, SB)],
                         sbuf.at[1 - pc], ssem.at[1 - pc])
        pltpu.async_copy(dst2.at[pl.ds(rbase + (ck + 1) * SB, SB)],
                         dbuf.at[1 - pc], ssem.at[1 - pc])

      @pl.when(gp >= 2)
      def _():
        pltpu.make_async_copy(msg.at[pp], agg_sh.at[scidx.at[pp]],
                              csem.at[pp]).wait()

      for i in range(C // L):
        scidx[pp, pl.ds(i * L, L)] = dbuf[ckp, rp, pl.ds(i * L, L)]

      JV = D // L

      def edge(e, _):
        eks = [ekb[pp, e, pl.ds(j * L, L)] for j in range(JV)]
        eqs = [eqb[pp, e, pl.ds(j * L, L)] for j in range(JV)]
        xs = [1.0 + jnp.minimum(a * q, 1e30) for a, q in zip(eks, eqs)]
        ys = [plsc.bitcast(magic - plsc.bitcast(x, jnp.int32), jnp.float32)
              for x in xs]
        for _ in range(2):
          ts = [x * y for x, y in zip(xs, ys)]
          us = [2.0 - t for t in ts]
          ys = [y * u for y, u in zip(ys, us)]
        vs = [vb[pp, e, pl.ds(j * L, L)] for j in range(JV)]
        ms = [y * v for y, v in zip(ys, vs)]
        for j in range(JV):
          msg[pp, e, pl.ds(j * L, L)] = ms[j]
        return 0

      lax.fori_loop(0, C, edge, 0)

      pltpu.async_copy(msg.at[pp], agg_sh.at[scidx.at[pp]], csem.at[pp],
                       add=True)

    return 0

  lax.fori_loop(0, TOTB + 1, gblk, 0)

  pltpu.make_async_copy(msg.at[0], agg_sh.at[scidx.at[0]], csem.at[0]).wait()
  pltpu.make_async_copy(msg.at[1], agg_sh.at[scidx.at[1]], csem.at[1]).wait()
  plsc.subcore_barrier()

  pltpu.sync_copy(agg_sh.at[pl.ds(sid * STRIPE, STRIPE)],
                  part_hbm.at[cid, pl.ds(sid * STRIPE, STRIPE)])


def _sc_edge(ek, eq, v, src2, dst2):
  mesh = plsc.VectorSubcoreMesh(
      core_axis_name="c", subcore_axis_name="s",
      num_cores=NC, num_subcores=NS)
  f = functools.partial(
      pl.kernel,
      out_type=jax.ShapeDtypeStruct((NC, NP, D), jnp.float32),
      mesh=mesh,
      compiler_params=pltpu.CompilerParams(needs_layout_passes=False),
      scratch_types=[
          pltpu.VMEM((2, SB, C), jnp.int32),
          pltpu.VMEM((2, SB, C), jnp.int32),
          pltpu.VMEM((2, C), jnp.int32),
          pltpu.VMEM((2, C, D), jnp.float32),
          pltpu.VMEM((2, C, D), jnp.float32),
          pltpu.VMEM((2, C, D), jnp.float32),
          pltpu.VMEM((2, C, D), jnp.float32),
          pltpu.VMEM_SHARED((NP, D), jnp.float32),
          pltpu.SemaphoreType.DMA((2,)),
          pltpu.SemaphoreType.DMA((2,)),
          pltpu.SemaphoreType.DMA((2,)),
      ],
  )(_sc_edge_kernel)
  return f(ek, eq, v, src2, dst2)


@jax.jit
def kernel(x, edge_index, edge_attr, Wk, bk, Wq, bq, Wv, bv, Wskip, bias):
  del edge_attr
  xp = jnp.pad(x, ((0, NP - N), (0, 0)))
  wt = jnp.stack([Wk.T, Wq.T, Wv.T, Wskip.T])
  b = jnp.stack([bk, bq, bv, bias])
  ek, eq, v, skip = _tc_proj(xp, wt, b)
  src = edge_index[0].astype(jnp.int32)
  dst = edge_index[1].astype(jnp.int32)
  pad = EB * C - E
  src2 = jnp.pad(src, (0, pad)).reshape(EB, C)
  dst2 = jnp.pad(dst, (0, pad), constant_values=N).reshape(EB, C)
  part = _sc_edge(ek, eq, v, src2, dst2)
  out = _tc_combine(part, skip)
  return out[:N]
